# baseline (speedup 1.0000x reference)
import functools

import jax
import jax.numpy as jnp
from jax import lax
from jax.experimental import pallas as pl
from jax.experimental.pallas import tpu as pltpu
from jax.experimental.pallas import tpu_sc as plsc

VOCAB = 1000000
D = 64
B = 16384
NC = 2
NS = 16
NW = NC * NS
BPW = B // NW
CH = 128
NCH = BPW // CH
LANES = 16
PR = 2 * D


@functools.cache
def _build_encoder():
    mesh = plsc.VectorSubcoreMesh(core_axis_name="c", subcore_axis_name="s")

    @functools.partial(
        pl.kernel,
        mesh=mesh,
        out_type=jax.ShapeDtypeStruct((B // 2, PR), jnp.float32),
        scratch_types=[
            pltpu.VMEM((NCH, CH), jnp.int32),
            pltpu.VMEM((NCH, CH), jnp.int32),
            pltpu.VMEM((2, CH, PR), jnp.float32),
            pltpu.VMEM((2, CH, PR), jnp.float32),
            pltpu.VMEM((BPW // 2, PR), jnp.float32),
            pltpu.SemaphoreType.DMA,
            pltpu.SemaphoreType.DMA,
        ],
        compiler_params=pltpu.CompilerParams(needs_layout_passes=False),
    )
    def _encode(idxp_hbm, hb_hbm, item_hbm, ent_hbm, out_hbm,
                idxp_v, hb_v, a_v, b_v, out_v, sem_a, sem_b):
        wid = lax.axis_index("s") * NC + lax.axis_index("c")

        pltpu.sync_copy(idxp_hbm.at[pl.ds(wid * NCH, NCH)], idxp_v)
        pltpu.sync_copy(hb_hbm.at[pl.ds(wid * NCH, NCH)], hb_v)

        def fire(j, buf):
            ca = pltpu.async_copy(item_hbm.at[idxp_v.at[j]], a_v.at[buf], sem_a)
            cb = pltpu.async_copy(ent_hbm.at[idxp_v.at[j]], b_v.at[buf], sem_b)
            return ca, cb

        lane = lax.iota(jnp.int32, LANES)

        def repack(j, buf):
            aj = a_v.at[buf]
            bj = b_v.at[buf]

            def row_body(r, carry):
                hb16 = plsc.load_gather(hb_v, [jnp.full((LANES,), j, jnp.int32),
                                               jnp.full((LANES,), r, jnp.int32)])
                rr = jnp.full((LANES,), r, jnp.int32)
                g = j * CH + r
                q = g // 2
                co = (g % 2) * D
                for cg in range(D // LANES):
                    col = hb16 + (cg * LANES) + lane
                    va = plsc.load_gather(aj, [rr, col])
                    vb = plsc.load_gather(bj, [rr, col])
                    out_v[q, pl.ds(co + cg * LANES, LANES)] = va + vb
                return carry

            lax.fori_loop(0, CH, row_body, 0)

        cops = fire(0, 0)
        for j in range(NCH):
            nxt = fire(j + 1, (j + 1) % 2) if j + 1 < NCH else None
            for c in cops:
                c.wait()
            repack(j, j % 2)
            cops = nxt

        pltpu.sync_copy(out_v, out_hbm.at[pl.ds(wid * (BPW // 2), BPW // 2)])

    return _encode


def kernel(batch_data, item_table, ent_table):
    idxp = (batch_data >> 1).reshape(NW * NCH, CH)
    hb = ((batch_data & 1) * D).reshape(NW * NCH, CH)
    item2 = item_table.reshape(VOCAB // 2, PR)
    ent2 = ent_table.reshape(VOCAB // 2, PR)
    out2 = _build_encoder()(idxp, hb, item2, ent2)
    return out2.reshape(B, D)


# speedup vs baseline: 1.0203x; 1.0203x over previous
import functools

import jax
import jax.numpy as jnp
from jax import lax
from jax.experimental import pallas as pl
from jax.experimental.pallas import tpu as pltpu
from jax.experimental.pallas import tpu_sc as plsc

VOCAB = 1000000
D = 64
B = 16384
NC = 2
NS = 16
NW = NC * NS
WIN = 128
NWINT = 7813
NWIN = 245
SPAD = 248
LANES = 16
STAGE = 128


@functools.cache
def _build_encoder():
    mesh = plsc.VectorSubcoreMesh(core_axis_name="c", subcore_axis_name="s")

    def full16(x):
        return jnp.full((LANES,), x, jnp.int32)

    @functools.partial(
        pl.kernel,
        mesh=mesh,
        out_type=jax.ShapeDtypeStruct((B, 2 * D), jnp.float32),
        scratch_types=[
            pltpu.VMEM((B,), jnp.int32),
            pltpu.VMEM((B,), jnp.int32),
            pltpu.VMEM((SPAD,), jnp.int32),
            pltpu.VMEM((2, D, WIN), jnp.float32),
            pltpu.VMEM((2, D, WIN), jnp.float32),
            pltpu.VMEM((STAGE, 2 * D), jnp.float32),
            pltpu.VMEM((STAGE,), jnp.int32),
            pltpu.SemaphoreType.DMA,
            pltpu.SemaphoreType.DMA,
            pltpu.SemaphoreType.DMA,
        ],
        compiler_params=pltpu.CompilerParams(needs_layout_passes=False),
    )
    def _encode(sv_hbm, perm_hbm, starts_hbm, itemT_hbm, entT_hbm, out_hbm,
                sv_v, perm_v, starts_v, ibuf, ebuf, stage, sb_v,
                sem_i, sem_e, sem_ld):
        lane = lax.iota(jnp.int32, LANES)
        wid = lax.axis_index("s") * NC + lax.axis_index("c")
        wbase = wid * NWIN
        nwin = jnp.where(wid == NW - 1, NWINT - (NW - 1) * NWIN, NWIN)

        pltpu.sync_copy(sv_hbm, sv_v)
        pltpu.sync_copy(perm_hbm, perm_v)
        pltpu.sync_copy(starts_hbm.at[pl.ds(wid * SPAD, SPAD)], starts_v)

        def fire(k, buf):
            pass

        def wait_window(buf):
            pass

        def load_window(k, buf):
            col = (wbase + k) * WIN
            pltpu.sync_copy(itemT_hbm.at[:, pl.ds(col, WIN)], ibuf.at[buf])
            pltpu.sync_copy(entT_hbm.at[:, pl.ds(col, WIN)], ebuf.at[buf])

        def extract(x16):
            return jnp.max(x16)

        lane0 = lane == 0

        def sb_write(pos, val16):
            plsc.store_scatter(sb_v, [full16(pos)], val16, mask=lane0)

        def flush():
            pltpu.sync_copy(stage, out_hbm.at[sb_v])

        def win_body(k, carry):
            gp, s = carry
            buf = 0
            buf16 = full16(buf)
            load_window(k, buf)
            e = extract(plsc.load_gather(starts_v, [full16(k + 1)]))

            def hit(t, gp):
                t16 = full16(t)
                v16 = plsc.load_gather(sv_v, [t16])
                lane16 = v16 & (WIN - 1)
                b16 = plsc.load_gather(perm_v, [t16])
                spm = gp & (STAGE - 1)
                for si in range(D // LANES):
                    rows = lane + si * LANES
                    va = plsc.load_gather(ibuf, [buf16, rows, lane16])
                    vb = plsc.load_gather(ebuf, [buf16, rows, lane16])
                    stage[spm, pl.ds(si * LANES, LANES)] = va + vb
                sb_write(spm, b16)
                gp1 = gp + 1

                @pl.when((gp1 & (STAGE - 1)) == 0)
                def _():
                    flush()

                return gp1

            gp = lax.fori_loop(s, e, hit, gp)
            return gp, e

        s0 = extract(plsc.load_gather(starts_v, [full16(0)]))
        gp, _ = lax.fori_loop(0, nwin, win_body, (jnp.int32(0), s0))

        rem = gp & (STAGE - 1)

        @pl.when(rem != 0)
        def _():
            t0 = s0 + gp - rem
            b016 = plsc.load_gather(perm_v, [full16(t0)])

            def pad(p, c):
                for si in range(D // LANES):
                    stage[p, pl.ds(si * LANES, LANES)] = \
                        stage[0, pl.ds(si * LANES, LANES)]
                sb_write(p, b016)
                return c

            lax.fori_loop(rem, STAGE, pad, 0)
            flush()

    return _encode


def kernel(batch_data, item_table, ent_table):
    perm = jnp.argsort(batch_data).astype(jnp.int32)
    sv = batch_data[perm]
    bounds = jnp.arange(NWINT + 1, dtype=jnp.int32) * WIN
    starts = jnp.searchsorted(sv, bounds, side="left").astype(jnp.int32)
    sidx = (jnp.arange(NW, dtype=jnp.int32)[:, None] * NWIN
            + jnp.arange(SPAD, dtype=jnp.int32)[None, :]).clip(0, NWINT)
    starts_pad = starts[sidx].reshape(-1)
    out = _build_encoder()(sv, perm, starts_pad, item_table.T, ent_table.T)
    return out[:, :D]
